# baseline (speedup 1.0000x reference)
import jax
import jax.numpy as jnp
from jax import lax
from jax.experimental import pallas as pl
from jax.experimental.pallas import tpu as pltpu
from jax.experimental.pallas import tpu_sc as plsc

N = 16384
L = 50
B_DIM = 16
FEAT = 16
OUT_W = B_DIM + FEAT

NC = 2
NS = 16
NW = NC * NS

TOTAL = N * L
G = 128
ROWS_PER_W = TOTAL // (NW * G)
CHUNK_ROWS = 10
C = CHUNK_ROWS * G
N_CHUNKS = ROWS_PER_W // CHUNK_ROWS
NBUF = 2
N_ROUNDS = N_CHUNKS // NBUF


def _sc_body(idx_hbm, feat_hbm, table_hbm, out_hbm,
             idx_v, rows_v, feat_v, *sems):
    gsem = sems[0:NBUF]
    fsem = sems[NBUF:2 * NBUF]
    wsem = sems[2 * NBUF:3 * NBUF]
    wid = lax.axis_index("s") * NC + lax.axis_index("c")
    row_base = wid * ROWS_PER_W
    base = row_base * G

    pltpu.sync_copy(idx_hbm.at[pl.ds(row_base, ROWS_PER_W)], idx_v)

    def issue(i, b):
        for j in range(CHUNK_ROWS):
            pltpu.async_copy(
                table_hbm.at[idx_v.at[i * CHUNK_ROWS + j]],
                rows_v.at[b, pl.ds(j * G, G)],
                gsem[b],
            )
        pltpu.async_copy(
            feat_hbm.at[pl.ds(base + i * C, C)], feat_v.at[b], fsem[b],
        )

    def drain(i, b):
        for j in range(CHUNK_ROWS):
            pltpu.make_async_copy(
                table_hbm.at[idx_v.at[i * CHUNK_ROWS + j]],
                rows_v.at[b, pl.ds(j * G, G)],
                gsem[b],
            ).wait()
        pltpu.make_async_copy(
            feat_hbm.at[pl.ds(base + i * C, C)], feat_v.at[b], fsem[b],
        ).wait()

    for b in range(NBUF):
        issue(b, b)

    def round_body(r, _):
        for b in range(NBUF):
            i = r * NBUF + b
            drain(i, b)
            w1 = pltpu.async_copy(
                rows_v.at[b],
                out_hbm.at[pl.ds(base + i * C, C), pl.ds(0, B_DIM)],
                wsem[b],
            )
            w2 = pltpu.async_copy(
                feat_v.at[b],
                out_hbm.at[pl.ds(base + i * C, C), pl.ds(B_DIM, FEAT)],
                wsem[b],
            )
            nxt = i + NBUF

            @pl.when(nxt < N_CHUNKS)
            def _():
                w1.wait()
                w2.wait()
                issue(nxt, b)

        return ()

    lax.fori_loop(0, N_ROUNDS, round_body, (), unroll=False)
    for b in range(NBUF):
        for _ in range(2):
            pltpu.make_async_copy(
                rows_v.at[b],
                out_hbm.at[pl.ds(base, C), pl.ds(0, B_DIM)],
                wsem[b],
            ).wait()


@jax.jit
def _run(idx2d, feat2d, table):
    kern = pl.kernel(
        _sc_body,
        out_type=jax.ShapeDtypeStruct((TOTAL, OUT_W), jnp.float32),
        mesh=plsc.VectorSubcoreMesh(
            core_axis_name="c", subcore_axis_name="s",
            num_cores=NC, num_subcores=NS,
        ),
        scratch_types=[
            pltpu.VMEM((ROWS_PER_W, G), jnp.int32),
            pltpu.VMEM((NBUF, C, B_DIM), jnp.float32),
            pltpu.VMEM((NBUF, C, FEAT), jnp.float32),
        ] + [pltpu.SemaphoreType.DMA] * (3 * NBUF),
        compiler_params=pltpu.CompilerParams(use_tc_tiling_on_sc=False),
    )
    return kern(idx2d, feat2d, table)


def kernel(segmentId, pathSegmentFeat, table):
    idx2d = segmentId.astype(jnp.int32).reshape(TOTAL // G, G)
    feat2d = pathSegmentFeat.reshape(TOTAL, FEAT)
    out = _run(idx2d, feat2d, table)
    return out.reshape(N, L, OUT_W)


# speedup vs baseline: 1.4259x; 1.4259x over previous
import jax
import jax.numpy as jnp
from jax import lax
from jax.experimental import pallas as pl
from jax.experimental.pallas import tpu as pltpu
from jax.experimental.pallas import tpu_sc as plsc

N = 16384
L = 50
B_DIM = 16
FEAT = 16
OUT_W = B_DIM + FEAT

NC = 2
NS = 16
NW = NC * NS

TOTAL = N * L
G = 128
ROWS_PER_W = TOTAL // (NW * G)
CHUNK_ROWS = 10
C = CHUNK_ROWS * G
N_CHUNKS = ROWS_PER_W // CHUNK_ROWS
NBUF = 2
N_ROUNDS = N_CHUNKS // NBUF


def _sc_body(idx_hbm, feat_hbm, table_hbm, oe_hbm, oo_hbm, out_hbm,
             idx_v, oe_v, oo_v, rows_v, feat_v, *sems):
    gsem = sems[0:NBUF]
    asem = sems[NBUF:2 * NBUF]
    ssem = sems[2 * NBUF:3 * NBUF]
    wid = lax.axis_index("s") * NC + lax.axis_index("c")
    row_base = wid * ROWS_PER_W
    base = row_base * G

    pltpu.sync_copy(idx_hbm.at[pl.ds(row_base, ROWS_PER_W)], idx_v)

    def issue(i, b):
        for j in range(CHUNK_ROWS):
            pltpu.async_copy(
                table_hbm.at[idx_v.at[i * CHUNK_ROWS + j]],
                rows_v.at[b, pl.ds(j * G, G)],
                gsem[b],
            )
        row_off = row_base + i * CHUNK_ROWS
        pltpu.async_copy(
            feat_hbm.at[pl.ds(base + i * C, C)], feat_v.at[b], asem[b])
        pltpu.async_copy(
            oe_hbm.at[pl.ds(row_off, CHUNK_ROWS)], oe_v.at[b], asem[b])
        pltpu.async_copy(
            oo_hbm.at[pl.ds(row_off, CHUNK_ROWS)], oo_v.at[b], asem[b])

    def drain_in(i, b):
        for j in range(CHUNK_ROWS):
            pltpu.make_async_copy(
                table_hbm.at[idx_v.at[i * CHUNK_ROWS + j]],
                rows_v.at[b, pl.ds(j * G, G)],
                gsem[b],
            ).wait()
        row_off = row_base + i * CHUNK_ROWS
        pltpu.make_async_copy(
            feat_hbm.at[pl.ds(base + i * C, C)], feat_v.at[b], asem[b]).wait()
        pltpu.make_async_copy(
            oe_hbm.at[pl.ds(row_off, CHUNK_ROWS)], oe_v.at[b], asem[b]).wait()
        pltpu.make_async_copy(
            oo_hbm.at[pl.ds(row_off, CHUNK_ROWS)], oo_v.at[b], asem[b]).wait()

    def scatter_out(b):
        for j in range(CHUNK_ROWS):
            pltpu.async_copy(
                rows_v.at[b, pl.ds(j * G, G)],
                out_hbm.at[oe_v.at[b, j]],
                ssem[b],
            )
            pltpu.async_copy(
                feat_v.at[b, pl.ds(j * G, G)],
                out_hbm.at[oo_v.at[b, j]],
                ssem[b],
            )

    def drain_out(b):
        for j in range(CHUNK_ROWS):
            pltpu.make_async_copy(
                rows_v.at[b, pl.ds(j * G, G)],
                out_hbm.at[oe_v.at[b, j]],
                ssem[b],
            ).wait()
            pltpu.make_async_copy(
                feat_v.at[b, pl.ds(j * G, G)],
                out_hbm.at[oo_v.at[b, j]],
                ssem[b],
            ).wait()

    for b in range(NBUF):
        issue(b, b)

    def round_body(r, _):
        for b in range(NBUF):
            i = r * NBUF + b
            drain_in(i, b)
            scatter_out(b)
            nxt = i + NBUF

            @pl.when(nxt < N_CHUNKS)
            def _():
                drain_out(b)
                issue(nxt, b)

        return ()

    lax.fori_loop(0, N_ROUNDS, round_body, (), unroll=False)
    for b in range(NBUF):
        drain_out(b)


@jax.jit
def _run(idx2d, feat2d, table, oe2d, oo2d):
    kern = pl.kernel(
        _sc_body,
        out_type=jax.ShapeDtypeStruct((2 * TOTAL, B_DIM), jnp.float32),
        mesh=plsc.VectorSubcoreMesh(
            core_axis_name="c", subcore_axis_name="s",
            num_cores=NC, num_subcores=NS,
        ),
        scratch_types=[
            pltpu.VMEM((ROWS_PER_W, G), jnp.int32),
            pltpu.VMEM((NBUF, CHUNK_ROWS, G), jnp.int32),
            pltpu.VMEM((NBUF, CHUNK_ROWS, G), jnp.int32),
            pltpu.VMEM((NBUF, C, B_DIM), jnp.float32),
            pltpu.VMEM((NBUF, C, FEAT), jnp.float32),
        ] + [pltpu.SemaphoreType.DMA] * (3 * NBUF),
        compiler_params=pltpu.CompilerParams(use_tc_tiling_on_sc=False),
    )
    return kern(idx2d, feat2d, table, oe2d, oo2d)


def kernel(segmentId, pathSegmentFeat, table):
    idx2d = segmentId.astype(jnp.int32).reshape(TOTAL // G, G)
    feat2d = pathSegmentFeat.reshape(TOTAL, FEAT)
    oe2d = (2 * jnp.arange(TOTAL, dtype=jnp.int32)).reshape(TOTAL // G, G)
    oo2d = oe2d + 1
    out = _run(idx2d, feat2d, table, oe2d, oo2d)
    return out.reshape(N, L, OUT_W)


# speedup vs baseline: 3.0115x; 2.1121x over previous
import jax
import jax.numpy as jnp
from jax import lax
from jax.experimental import pallas as pl
from jax.experimental.pallas import tpu as pltpu
from jax.experimental.pallas import tpu_sc as plsc

N = 16384
L = 50
B_DIM = 16
FEAT = 16
OUT_W = B_DIM + FEAT

NC = 2
NS = 16
NW = NC * NS
LANE = 128
NT = N // LANE
TPW = NT // NW
CH = OUT_W // 8
CHF = FEAT // 8


def _transpose_rows(rows2d, kbuf_t):
    base = lax.iota(jnp.int32, 16)
    for k in range(LANE // 16):
        nl_idx = base + (16 * k)
        for c in range(B_DIM):
            c_idx = jnp.full((16,), c, jnp.int32)
            val = plsc.load_gather(rows2d, [nl_idx, c_idx])
            kbuf_t[c // 8, c % 8, pl.ds(16 * k, 16)] = val


def _sc_body(idx_hbm, feat_hbm, table_hbm, out_hbm,
             idx_v, rows_v, kbuf, *sems):
    gsem = sems[0:TPW]
    fsem = sems[TPW:2 * TPW]
    wsem = sems[2 * TPW:3 * TPW]
    wid = lax.axis_index("s") * NC + lax.axis_index("c")
    nt0 = wid * TPW

    pltpu.sync_copy(idx_hbm.at[:, pl.ds(nt0, TPW)], idx_v)

    def issue_gather(l, t):
        pltpu.async_copy(
            table_hbm.at[idx_v.at[l, t]], rows_v.at[t], gsem[t])

    def drain_gather(l, t):
        pltpu.make_async_copy(
            table_hbm.at[idx_v.at[l, t]], rows_v.at[t], gsem[t]).wait()

    for t in range(TPW):
        issue_gather(0, t)

    def l_body(l, _):
        for t in range(TPW):
            nt = nt0 + t

            @pl.when(l > 0)
            def _():
                for ch in range(CH):
                    pltpu.make_async_copy(
                        kbuf.at[t, ch], out_hbm.at[l, ch, nt], wsem[t]
                    ).wait()

            drain_gather(l, t)
            for chf in range(CHF):
                pltpu.async_copy(
                    feat_hbm.at[l, chf, nt], kbuf.at[t, B_DIM // 8 + chf],
                    fsem[t])
            _transpose_rows(rows_v.at[t], kbuf.at[t])

            @pl.when(l + 1 < L)
            def _():
                issue_gather(l + 1, t)

            for chf in range(CHF):
                pltpu.make_async_copy(
                    feat_hbm.at[l, chf, nt], kbuf.at[t, B_DIM // 8 + chf],
                    fsem[t]).wait()
            for ch in range(CH):
                pltpu.async_copy(
                    kbuf.at[t, ch], out_hbm.at[l, ch, nt], wsem[t])
        return ()

    lax.fori_loop(0, L, l_body, (), unroll=False)
    for t in range(TPW):
        for ch in range(CH):
            pltpu.make_async_copy(
                kbuf.at[t, ch], out_hbm.at[L - 1, ch, nt0 + t], wsem[t]
            ).wait()


@jax.jit
def _run(idx3, feat5, table):
    kern = pl.kernel(
        _sc_body,
        out_type=jax.ShapeDtypeStruct((L, CH, NT, 8, LANE), jnp.float32),
        mesh=plsc.VectorSubcoreMesh(
            core_axis_name="c", subcore_axis_name="s",
            num_cores=NC, num_subcores=NS,
        ),
        scratch_types=[
            pltpu.VMEM((L, TPW, LANE), jnp.int32),
            pltpu.VMEM((TPW, LANE, B_DIM), jnp.float32),
            pltpu.VMEM((TPW, CH, 8, LANE), jnp.float32),
        ] + [pltpu.SemaphoreType.DMA] * (3 * TPW),
        compiler_params=pltpu.CompilerParams(
            use_tc_tiling_on_sc=False, needs_layout_passes=False),
    )
    return kern(idx3, feat5, table)


def kernel(segmentId, pathSegmentFeat, table):
    idx3 = segmentId.astype(jnp.int32).reshape(N, L).T.reshape(L, NT, LANE)
    feat5 = (pathSegmentFeat.transpose(1, 2, 0)
             .reshape(L, CHF, 8, NT, LANE).transpose(0, 1, 3, 2, 4))
    out = _run(idx3, feat5, table)
    return out.transpose(2, 4, 0, 1, 3).reshape(N, L, OUT_W)
